# Initial kernel scaffold; baseline (speedup 1.0000x reference)
import functools

import jax
import jax.numpy as jnp
from jax import lax
from jax.experimental import pallas as pl
from jax.experimental.pallas import tpu as pltpu
from jax.experimental.pallas import tpu_sc as plsc

N = 10000
E = 320000
D_IN = 128
D_H = 256
D_OUT = 128
G = 128

NC = 2
NS = 16
NW = NC * NS

NP = 10240
ROWS_W = NP // NW
ROWS_T = NP // NS

EC = 128
KW = 79
EP = NW * KW * EC


def _sc_mesh():
    return plsc.VectorSubcoreMesh(core_axis_name="c", subcore_axis_name="s")


def _gather_body(embed_hbm, nid_hbm, x_hbm, idx_v, rows_v, sem):
    wid = lax.axis_index("s") * NC + lax.axis_index("c")
    base = wid * ROWS_W
    pltpu.sync_copy(nid_hbm.at[pl.ds(base, ROWS_W)], idx_v)
    descs = []
    for off, sz in ((0, 128), (128, 128), (256, 64)):
        descs.append(
            pltpu.async_copy(
                embed_hbm.at[idx_v.at[pl.ds(off, sz)]],
                rows_v.at[pl.ds(off, sz)],
                sem,
            )
        )
    for d in descs:
        d.wait()
    pltpu.sync_copy(rows_v, x_hbm.at[pl.ds(base, ROWS_W)])


def _gather_call(embed, nid_p):
    k = pl.kernel(
        _gather_body,
        out_type=jax.ShapeDtypeStruct((NP, D_IN), jnp.float32),
        mesh=_sc_mesh(),
        scratch_types=[
            pltpu.VMEM((ROWS_W,), jnp.int32),
            pltpu.VMEM((ROWS_W, D_IN), jnp.float32),
            pltpu.SemaphoreType.DMA,
        ],
    )
    return k(embed, nid_p)


def _edge_body(x_hbm, src_hbm, dst_hbm, zeros_hbm, agg_hbm,
               src_v, dst_v, rows_v, sem, agg_sh):
    cid = lax.axis_index("c")
    sid = lax.axis_index("s")
    wid = sid * NC + cid
    pltpu.sync_copy(zeros_hbm, agg_sh.at[pl.ds(sid * ROWS_T, ROWS_T)])
    pltpu.sync_copy(src_hbm.at[pl.ds(wid * KW, KW)], src_v)
    pltpu.sync_copy(dst_hbm.at[pl.ds(wid * KW, KW)], dst_v)
    plsc.subcore_barrier()

    def body(j, carry):
        pltpu.async_copy(x_hbm.at[src_v.at[j]], rows_v, sem).wait()
        pltpu.sync_copy(rows_v, agg_sh.at[dst_v.at[j]], add=True)
        return carry

    lax.fori_loop(0, KW, body, 0)
    plsc.subcore_barrier()
    pltpu.sync_copy(
        agg_sh.at[pl.ds(sid * ROWS_T, ROWS_T)],
        agg_hbm.at[cid, pl.ds(sid * ROWS_T, ROWS_T)],
    )


def _edge_call(x, src2d, dst2d, zeros):
    k = pl.kernel(
        _edge_body,
        out_type=jax.ShapeDtypeStruct((NC, NP, D_IN), jnp.float32),
        mesh=_sc_mesh(),
        scratch_types=[
            pltpu.VMEM((KW, EC), jnp.int32),
            pltpu.VMEM((KW, EC), jnp.int32),
            pltpu.VMEM((EC, D_IN), jnp.float32),
            pltpu.SemaphoreType.DMA,
            pltpu.VMEM_SHARED((NP, D_IN), jnp.float32),
        ],
    )
    return k(x, src2d, dst2d, zeros)


def _bn_cols(h, g, b):
    mu = jnp.mean(h, axis=0, keepdims=True)
    var = jnp.mean((h - mu) * (h - mu), axis=0, keepdims=True)
    return (h - mu) * lax.rsqrt(var + 1e-5) * g + b


def _mlp_body(x_ref, agg_ref, batch_ref,
              w1_ref, b1_ref, g1_ref, be1_ref,
              w2_ref, b2_ref, gbn_ref, bbn_ref,
              wp0_ref, bp0_ref, wp1_ref, bp1_ref,
              out_ref):
    x = x_ref[0:N, :]
    h = x + agg_ref[0, 0:N, :] + agg_ref[1, 0:N, :]
    h1 = jnp.dot(h, w1_ref[...], preferred_element_type=jnp.float32) + b1_ref[...]
    h1 = _bn_cols(h1, g1_ref[...], be1_ref[...])
    h1 = jnp.maximum(h1, 0.0)
    h2 = jnp.dot(h1, w2_ref[...], preferred_element_type=jnp.float32) + b2_ref[...]
    h2 = _bn_cols(h2, gbn_ref[...], bbn_ref[...])
    h2 = jnp.maximum(h2, 0.0)
    score = (jnp.dot(x, wp0_ref[...], preferred_element_type=jnp.float32)
             + bp0_ref[...]
             + jnp.dot(h2, wp1_ref[...], preferred_element_type=jnp.float32)
             + bp1_ref[...])
    onehot = (batch_ref[...] ==
              lax.broadcasted_iota(jnp.int32, (N, G), 1)).astype(jnp.float32)
    sums = lax.dot_general(onehot, score,
                           dimension_numbers=(((0,), (0,)), ((), ())),
                           preferred_element_type=jnp.float32)
    counts = jnp.sum(onehot, axis=0)
    out_ref[...] = sums / jnp.maximum(counts, 1.0)[:, None]


def _mlp_call(x, agg, batch2d, W1, b1, g1, be1, W2, b2, g_bn, b_bn,
              Wp0, bp0, Wp1, bp1):
    return pl.pallas_call(
        _mlp_body,
        out_shape=jax.ShapeDtypeStruct((G, D_OUT), jnp.float32),
    )(x, agg, batch2d,
      W1, b1.reshape(1, -1), g1.reshape(1, -1), be1.reshape(1, -1),
      W2, b2.reshape(1, -1), g_bn.reshape(1, -1), b_bn.reshape(1, -1),
      Wp0, bp0.reshape(1, -1), Wp1, bp1.reshape(1, -1))


def kernel(node_ids, edge_index, batch, embed, W1, b1, g1, be1, W2, b2,
           g_bn, b_bn, Wp0, bp0, Wp1, bp1):
    nid_p = jnp.zeros((NP,), jnp.int32).at[:N].set(node_ids.astype(jnp.int32))
    src = edge_index[0].astype(jnp.int32)
    dst = edge_index[1].astype(jnp.int32)
    src2d = jnp.zeros((EP,), jnp.int32).at[:E].set(src).reshape(NW * KW, EC)
    dst2d = (jnp.full((EP,), NP - 1, jnp.int32).at[:E].set(dst)
             .reshape(NW * KW, EC))
    zeros = jnp.zeros((ROWS_T, D_IN), jnp.float32)

    x = _gather_call(embed, nid_p)
    agg = _edge_call(x, src2d, dst2d, zeros)
    batch2d = batch.astype(jnp.int32).reshape(N, 1)
    return _mlp_call(x, agg, batch2d, W1, b1, g1, be1, W2, b2,
                     g_bn, b_bn, Wp0, bp0, Wp1, bp1)


# speedup vs baseline: 4.0309x; 4.0309x over previous
import functools

import jax
import jax.numpy as jnp
from jax import lax
from jax.experimental import pallas as pl
from jax.experimental.pallas import tpu as pltpu
from jax.experimental.pallas import tpu_sc as plsc

N = 10000
E = 320000
D_IN = 128
D_H = 256
D_OUT = 128
G = 128

NC = 2
NS = 16
NW = NC * NS

NP = 10240
ROWS_W = NP // NW
ROWS_T = NP // NS

EC = 128
KW = 80
EP = NW * KW * EC


def _sc_mesh():
    return plsc.VectorSubcoreMesh(core_axis_name="c", subcore_axis_name="s")


def _gather_body(embed_hbm, nid_hbm, x_hbm, idx_v, rows_v, sem):
    wid = lax.axis_index("s") * NC + lax.axis_index("c")
    base = wid * ROWS_W
    pltpu.sync_copy(nid_hbm.at[pl.ds(base, ROWS_W)], idx_v)
    descs = []
    for off, sz in ((0, 128), (128, 128), (256, 64)):
        descs.append(
            pltpu.async_copy(
                embed_hbm.at[idx_v.at[pl.ds(off, sz)]],
                rows_v.at[pl.ds(off, sz)],
                sem,
            )
        )
    for d in descs:
        d.wait()
    pltpu.sync_copy(rows_v, x_hbm.at[pl.ds(base, ROWS_W)])


def _gather_call(embed, nid_p):
    k = pl.kernel(
        _gather_body,
        out_type=jax.ShapeDtypeStruct((NP, D_IN), jnp.float32),
        mesh=_sc_mesh(),
        scratch_types=[
            pltpu.VMEM((ROWS_W,), jnp.int32),
            pltpu.VMEM((ROWS_W, D_IN), jnp.float32),
            pltpu.SemaphoreType.DMA,
        ],
    )
    return k(embed, nid_p)


def _edge_body(x_hbm, src_hbm, dst_hbm, zeros_hbm, agg_hbm,
               src_v, dst_v, rows_v, sem, agg_sh):
    cid = lax.axis_index("c")
    sid = lax.axis_index("s")
    wid = sid * NC + cid
    pltpu.sync_copy(zeros_hbm, agg_sh.at[pl.ds(sid * ROWS_T, ROWS_T)])
    pltpu.sync_copy(src_hbm.at[pl.ds(wid * KW, KW)], src_v)
    pltpu.sync_copy(dst_hbm.at[pl.ds(wid * KW, KW)], dst_v)
    plsc.subcore_barrier()

    def body(j, carry):
        pltpu.async_copy(x_hbm.at[src_v.at[j]], rows_v, sem).wait()
        pltpu.sync_copy(rows_v, agg_sh.at[dst_v.at[j]], add=True)
        return carry

    lax.fori_loop(0, KW, body, 0)
    plsc.subcore_barrier()
    pltpu.sync_copy(
        agg_sh.at[pl.ds(sid * ROWS_T, ROWS_T)],
        agg_hbm.at[cid, pl.ds(sid * ROWS_T, ROWS_T)],
    )


def _edge_call(x, src2d, dst2d, zeros):
    k = pl.kernel(
        _edge_body,
        out_type=jax.ShapeDtypeStruct((NC, NP, D_IN), jnp.float32),
        mesh=_sc_mesh(),
        scratch_types=[
            pltpu.VMEM((KW, EC), jnp.int32),
            pltpu.VMEM((KW, EC), jnp.int32),
            pltpu.VMEM((EC, D_IN), jnp.float32),
            pltpu.SemaphoreType.DMA,
            pltpu.VMEM_SHARED((NP, D_IN), jnp.float32),
        ],
    )
    return k(x, src2d, dst2d, zeros)


def _bn_cols(h, g, b):
    mu = jnp.mean(h, axis=0, keepdims=True)
    var = jnp.mean((h - mu) * (h - mu), axis=0, keepdims=True)
    return (h - mu) * lax.rsqrt(var + 1e-5) * g + b


def _mlp_body(x_ref, agg_ref, batch_ref,
              w1_ref, b1_ref, g1_ref, be1_ref,
              w2_ref, b2_ref, gbn_ref, bbn_ref,
              wp0_ref, bp0_ref, wp1_ref, bp1_ref,
              out_ref):
    x = x_ref[0:N, :]
    h = x + agg_ref[0, 0:N, :] + agg_ref[1, 0:N, :]
    h1 = jnp.dot(h, w1_ref[...], preferred_element_type=jnp.float32) + b1_ref[...]
    h1 = _bn_cols(h1, g1_ref[...], be1_ref[...])
    h1 = jnp.maximum(h1, 0.0)
    h2 = jnp.dot(h1, w2_ref[...], preferred_element_type=jnp.float32) + b2_ref[...]
    h2 = _bn_cols(h2, gbn_ref[...], bbn_ref[...])
    h2 = jnp.maximum(h2, 0.0)
    score = (jnp.dot(x, wp0_ref[...], preferred_element_type=jnp.float32)
             + bp0_ref[...]
             + jnp.dot(h2, wp1_ref[...], preferred_element_type=jnp.float32)
             + bp1_ref[...])
    onehot = (batch_ref[...] ==
              lax.broadcasted_iota(jnp.int32, (N, G), 1)).astype(jnp.float32)
    sums = lax.dot_general(onehot, score,
                           dimension_numbers=(((0,), (0,)), ((), ())),
                           preferred_element_type=jnp.float32)
    counts = jnp.sum(onehot, axis=0)
    out_ref[...] = sums / jnp.maximum(counts, 1.0)[:, None]


def _mlp_call(x, agg, batch2d, W1, b1, g1, be1, W2, b2, g_bn, b_bn,
              Wp0, bp0, Wp1, bp1):
    return pl.pallas_call(
        _mlp_body,
        out_shape=jax.ShapeDtypeStruct((G, D_OUT), jnp.float32),
    )(x, agg, batch2d,
      W1, b1.reshape(1, -1), g1.reshape(1, -1), be1.reshape(1, -1),
      W2, b2.reshape(1, -1), g_bn.reshape(1, -1), b_bn.reshape(1, -1),
      Wp0, bp0.reshape(1, -1), Wp1, bp1.reshape(1, -1))


def kernel(node_ids, edge_index, batch, embed, W1, b1, g1, be1, W2, b2,
           g_bn, b_bn, Wp0, bp0, Wp1, bp1):
    nid_p = jnp.zeros((NP,), jnp.int32).at[:N].set(node_ids.astype(jnp.int32))
    src = edge_index[0].astype(jnp.int32)
    dst = edge_index[1].astype(jnp.int32)
    src2d = jnp.zeros((EP,), jnp.int32).at[:E].set(src).reshape(NW * KW, EC)
    dst2d = (jnp.full((EP,), NP - 1, jnp.int32).at[:E].set(dst)
             .reshape(NW * KW, EC))
    zeros = jnp.zeros((ROWS_T, D_IN), jnp.float32)

    x = _gather_call(embed, nid_p)
    agg = _edge_call(x, src2d, dst2d, zeros)
    batch2d = batch.astype(jnp.int32).reshape(N, 1)
    return _mlp_call(x, agg, batch2d, W1, b1, g1, be1, W2, b2,
                     g_bn, b_bn, Wp0, bp0, Wp1, bp1)


# speedup vs baseline: 4.5346x; 1.1250x over previous
import functools

import jax
import jax.numpy as jnp
from jax import lax
from jax.experimental import pallas as pl
from jax.experimental.pallas import tpu as pltpu
from jax.experimental.pallas import tpu_sc as plsc

N = 10000
E = 320000
D_IN = 128
D_H = 256
D_OUT = 128
G = 128

NC = 2
NS = 16
NW = NC * NS

NP = 10240
ROWS_W = NP // NW
ROWS_T = NP // NS

EC = 128
KW = 80
EP = NW * KW * EC


def _sc_mesh():
    return plsc.VectorSubcoreMesh(core_axis_name="c", subcore_axis_name="s")


def _gather_body(embed_hbm, nid_hbm, x_hbm, idx_v, rows_v, sem):
    wid = lax.axis_index("s") * NC + lax.axis_index("c")
    base = wid * ROWS_W
    pltpu.sync_copy(nid_hbm.at[pl.ds(base, ROWS_W)], idx_v)
    descs = []
    for off, sz in ((0, 128), (128, 128), (256, 64)):
        descs.append(
            pltpu.async_copy(
                embed_hbm.at[idx_v.at[pl.ds(off, sz)]],
                rows_v.at[pl.ds(off, sz)],
                sem,
            )
        )
    for d in descs:
        d.wait()
    pltpu.sync_copy(rows_v, x_hbm.at[pl.ds(base, ROWS_W)])


def _gather_call(embed, nid_p):
    k = pl.kernel(
        _gather_body,
        out_type=jax.ShapeDtypeStruct((NP, D_IN), jnp.float32),
        mesh=_sc_mesh(),
        scratch_types=[
            pltpu.VMEM((ROWS_W,), jnp.int32),
            pltpu.VMEM((ROWS_W, D_IN), jnp.float32),
            pltpu.SemaphoreType.DMA,
        ],
    )
    return k(embed, nid_p)


KH = KW // 2


def _edge_body(x_hbm, src_hbm, dst_hbm, zeros_hbm, agg_hbm,
               src_v, dst_v, buf0, buf1, gsem, ssem, agg_sh):
    cid = lax.axis_index("c")
    sid = lax.axis_index("s")
    wid = sid * NC + cid
    pltpu.sync_copy(zeros_hbm, agg_sh.at[pl.ds(sid * ROWS_T, ROWS_T)])
    plsc.subcore_barrier()

    def drain_scatter(buf):
        pltpu.make_async_copy(buf, agg_sh.at[dst_v.at[0]], ssem).wait()

    for half in range(2):
        pltpu.sync_copy(src_hbm.at[pl.ds(wid * KW + half * KH, KH)], src_v)
        pltpu.sync_copy(dst_hbm.at[pl.ds(wid * KW + half * KH, KH)], dst_v)

        def body(i, carry):
            c0 = 2 * i

            @pl.when(i > 0)
            def _():
                drain_scatter(buf0)

            g0 = pltpu.async_copy(x_hbm.at[src_v.at[c0]], buf0, gsem)

            @pl.when(i > 0)
            def _():
                drain_scatter(buf1)

            g1 = pltpu.async_copy(x_hbm.at[src_v.at[c0 + 1]], buf1, gsem)
            g0.wait()
            pltpu.async_copy(buf0, agg_sh.at[dst_v.at[c0]], ssem, add=True)
            g1.wait()
            pltpu.async_copy(buf1, agg_sh.at[dst_v.at[c0 + 1]], ssem,
                             add=True)
            return carry

        lax.fori_loop(0, KH // 2, body, 0)
        drain_scatter(buf0)
        drain_scatter(buf1)
    plsc.subcore_barrier()
    pltpu.sync_copy(
        agg_sh.at[pl.ds(sid * ROWS_T, ROWS_T)],
        agg_hbm.at[cid, pl.ds(sid * ROWS_T, ROWS_T)],
    )


def _edge_call(x, src2d, dst2d, zeros):
    k = pl.kernel(
        _edge_body,
        out_type=jax.ShapeDtypeStruct((NC, NP, D_IN), jnp.float32),
        mesh=_sc_mesh(),
        scratch_types=[
            pltpu.VMEM((KH, EC), jnp.int32),
            pltpu.VMEM((KH, EC), jnp.int32),
            pltpu.VMEM((EC, D_IN), jnp.float32),
            pltpu.VMEM((EC, D_IN), jnp.float32),
            pltpu.SemaphoreType.DMA,
            pltpu.SemaphoreType.DMA,
            pltpu.VMEM_SHARED((NP, D_IN), jnp.float32),
        ],
    )
    return k(x, src2d, dst2d, zeros)


def _bn_cols(h, g, b):
    mu = jnp.mean(h, axis=0, keepdims=True)
    var = jnp.mean((h - mu) * (h - mu), axis=0, keepdims=True)
    return (h - mu) * lax.rsqrt(var + 1e-5) * g + b


def _mlp_body(x_ref, agg_ref, batch_ref,
              w1_ref, b1_ref, g1_ref, be1_ref,
              w2_ref, b2_ref, gbn_ref, bbn_ref,
              wp0_ref, bp0_ref, wp1_ref, bp1_ref,
              out_ref):
    x = x_ref[0:N, :]
    h = x + agg_ref[0, 0:N, :] + agg_ref[1, 0:N, :]
    h1 = jnp.dot(h, w1_ref[...], preferred_element_type=jnp.float32) + b1_ref[...]
    h1 = _bn_cols(h1, g1_ref[...], be1_ref[...])
    h1 = jnp.maximum(h1, 0.0)
    h2 = jnp.dot(h1, w2_ref[...], preferred_element_type=jnp.float32) + b2_ref[...]
    h2 = _bn_cols(h2, gbn_ref[...], bbn_ref[...])
    h2 = jnp.maximum(h2, 0.0)
    score = (jnp.dot(x, wp0_ref[...], preferred_element_type=jnp.float32)
             + bp0_ref[...]
             + jnp.dot(h2, wp1_ref[...], preferred_element_type=jnp.float32)
             + bp1_ref[...])
    onehot = (batch_ref[...] ==
              lax.broadcasted_iota(jnp.int32, (N, G), 1)).astype(jnp.float32)
    sums = lax.dot_general(onehot, score,
                           dimension_numbers=(((0,), (0,)), ((), ())),
                           preferred_element_type=jnp.float32)
    counts = jnp.sum(onehot, axis=0)
    out_ref[...] = sums / jnp.maximum(counts, 1.0)[:, None]


def _mlp_call(x, agg, batch2d, W1, b1, g1, be1, W2, b2, g_bn, b_bn,
              Wp0, bp0, Wp1, bp1):
    return pl.pallas_call(
        _mlp_body,
        out_shape=jax.ShapeDtypeStruct((G, D_OUT), jnp.float32),
    )(x, agg, batch2d,
      W1, b1.reshape(1, -1), g1.reshape(1, -1), be1.reshape(1, -1),
      W2, b2.reshape(1, -1), g_bn.reshape(1, -1), b_bn.reshape(1, -1),
      Wp0, bp0.reshape(1, -1), Wp1, bp1.reshape(1, -1))


def kernel(node_ids, edge_index, batch, embed, W1, b1, g1, be1, W2, b2,
           g_bn, b_bn, Wp0, bp0, Wp1, bp1):
    nid_p = jnp.zeros((NP,), jnp.int32).at[:N].set(node_ids.astype(jnp.int32))
    src = edge_index[0].astype(jnp.int32)
    dst = edge_index[1].astype(jnp.int32)
    src2d = jnp.zeros((EP,), jnp.int32).at[:E].set(src).reshape(NW * KW, EC)
    dst2d = (jnp.full((EP,), NP - 1, jnp.int32).at[:E].set(dst)
             .reshape(NW * KW, EC))
    zeros = jnp.zeros((ROWS_T, D_IN), jnp.float32)

    x = _gather_call(embed, nid_p)
    agg = _edge_call(x, src2d, dst2d, zeros)
    batch2d = batch.astype(jnp.int32).reshape(N, 1)
    return _mlp_call(x, agg, batch2d, W1, b1, g1, be1, W2, b2,
                     g_bn, b_bn, Wp0, bp0, Wp1, bp1)


# speedup vs baseline: 10.8675x; 2.3966x over previous
import functools

import jax
import jax.numpy as jnp
from jax import lax
from jax.experimental import pallas as pl
from jax.experimental.pallas import tpu as pltpu
from jax.experimental.pallas import tpu_sc as plsc

N = 10000
E = 320000
D_IN = 128
D_H = 256
D_OUT = 128
G = 128

NC = 2
NS = 16
NW = NC * NS

NP = 10240
ROWS_W = NP // NW
ROWS_T = NP // NS

EC = 128
KW = 80
EP = NW * KW * EC


def _sc_mesh():
    return plsc.VectorSubcoreMesh(core_axis_name="c", subcore_axis_name="s")


def _gather_body(embed_hbm, nid_hbm, x_hbm, idx_v, rows_v, sem):
    wid = lax.axis_index("s") * NC + lax.axis_index("c")
    base = wid * ROWS_W
    pltpu.sync_copy(nid_hbm.at[pl.ds(base, ROWS_W)], idx_v)
    descs = []
    for off, sz in ((0, 128), (128, 128), (256, 64)):
        descs.append(
            pltpu.async_copy(
                embed_hbm.at[idx_v.at[pl.ds(off, sz)]],
                rows_v.at[pl.ds(off, sz)],
                sem,
            )
        )
    for d in descs:
        d.wait()
    pltpu.sync_copy(rows_v, x_hbm.at[pl.ds(base, ROWS_W)])


def _gather_call(embed, nid_p):
    k = pl.kernel(
        _gather_body,
        out_type=jax.ShapeDtypeStruct((NP, D_IN), jnp.float32),
        mesh=_sc_mesh(),
        scratch_types=[
            pltpu.VMEM((ROWS_W,), jnp.int32),
            pltpu.VMEM((ROWS_W, D_IN), jnp.float32),
            pltpu.SemaphoreType.DMA,
        ],
    )
    return k(embed, nid_p)


KH = KW // 2


def _edge_body(x_hbm, src_hbm, dst_hbm, agg_hbm,
               src_v, dst_v, buf0, buf1, gsem, ssem, agg_sh):
    cid = lax.axis_index("c")
    sid = lax.axis_index("s")
    wid = sid * NC + cid

    zv = jnp.zeros((16,), jnp.float32)

    def zr(i, carry):
        buf0[i // 8, pl.ds((i % 8) * 16, 16)] = zv
        return carry

    lax.fori_loop(0, EC * 8, zr, 0)
    for c in range(ROWS_T // EC):
        pltpu.sync_copy(buf0, agg_sh.at[pl.ds(sid * ROWS_T + c * EC, EC)])
    plsc.subcore_barrier()

    def drain_scatter(buf):
        pltpu.make_async_copy(buf, agg_sh.at[dst_v.at[0]], ssem).wait()

    for half in range(2):
        pltpu.sync_copy(src_hbm.at[pl.ds(wid * KW + half * KH, KH)], src_v)
        pltpu.sync_copy(dst_hbm.at[pl.ds(wid * KW + half * KH, KH)], dst_v)

        def body(i, carry):
            c0 = 2 * i

            @pl.when(i > 0)
            def _():
                drain_scatter(buf0)

            g0 = pltpu.async_copy(x_hbm.at[src_v.at[c0]], buf0, gsem)

            @pl.when(i > 0)
            def _():
                drain_scatter(buf1)

            g1 = pltpu.async_copy(x_hbm.at[src_v.at[c0 + 1]], buf1, gsem)
            g0.wait()
            pltpu.async_copy(buf0, agg_sh.at[dst_v.at[c0]], ssem, add=True)
            g1.wait()
            pltpu.async_copy(buf1, agg_sh.at[dst_v.at[c0 + 1]], ssem,
                             add=True)
            return carry

        lax.fori_loop(0, KH // 2, body, 0)
        drain_scatter(buf0)
        drain_scatter(buf1)
    plsc.subcore_barrier()
    pltpu.sync_copy(
        agg_sh.at[pl.ds(sid * ROWS_T, ROWS_T)],
        agg_hbm.at[cid, pl.ds(sid * ROWS_T, ROWS_T)],
    )


def _edge_call(x, src2d, dst2d):
    k = pl.kernel(
        _edge_body,
        out_type=jax.ShapeDtypeStruct((NC, NP, D_IN), jnp.float32),
        mesh=_sc_mesh(),
        scratch_types=[
            pltpu.VMEM((KH, EC), jnp.int32),
            pltpu.VMEM((KH, EC), jnp.int32),
            pltpu.VMEM((EC, D_IN), jnp.float32),
            pltpu.VMEM((EC, D_IN), jnp.float32),
            pltpu.SemaphoreType.DMA,
            pltpu.SemaphoreType.DMA,
            pltpu.VMEM_SHARED((NP, D_IN), jnp.float32),
        ],
    )
    return k(x, src2d, dst2d)


def _bn_cols(h, g, b):
    mu = jnp.mean(h, axis=0, keepdims=True)
    var = jnp.mean((h - mu) * (h - mu), axis=0, keepdims=True)
    return (h - mu) * lax.rsqrt(var + 1e-5) * g + b


def _mlp_body(x_ref, agg_ref, batch_ref,
              w1_ref, b1_ref, g1_ref, be1_ref,
              w2_ref, b2_ref, gbn_ref, bbn_ref,
              wp0_ref, bp0_ref, wp1_ref, bp1_ref,
              out_ref):
    x = x_ref[0:N, :]
    h = x + agg_ref[0, 0:N, :] + agg_ref[1, 0:N, :]
    h1 = jnp.dot(h, w1_ref[...], preferred_element_type=jnp.float32) + b1_ref[...]
    h1 = _bn_cols(h1, g1_ref[...], be1_ref[...])
    h1 = jnp.maximum(h1, 0.0)
    h2 = jnp.dot(h1, w2_ref[...], preferred_element_type=jnp.float32) + b2_ref[...]
    h2 = _bn_cols(h2, gbn_ref[...], bbn_ref[...])
    h2 = jnp.maximum(h2, 0.0)
    score = (jnp.dot(x, wp0_ref[...], preferred_element_type=jnp.float32)
             + bp0_ref[...]
             + jnp.dot(h2, wp1_ref[...], preferred_element_type=jnp.float32)
             + bp1_ref[...])
    onehot = (batch_ref[...] ==
              lax.broadcasted_iota(jnp.int32, (N, G), 1)).astype(jnp.float32)
    sums = lax.dot_general(onehot, score,
                           dimension_numbers=(((0,), (0,)), ((), ())),
                           preferred_element_type=jnp.float32)
    counts = jnp.sum(onehot, axis=0)
    out_ref[...] = sums / jnp.maximum(counts, 1.0)[:, None]


def _mlp_call(x, agg, batch2d, W1, b1, g1, be1, W2, b2, g_bn, b_bn,
              Wp0, bp0, Wp1, bp1):
    return pl.pallas_call(
        _mlp_body,
        out_shape=jax.ShapeDtypeStruct((G, D_OUT), jnp.float32),
    )(x, agg, batch2d,
      W1, b1.reshape(1, -1), g1.reshape(1, -1), be1.reshape(1, -1),
      W2, b2.reshape(1, -1), g_bn.reshape(1, -1), b_bn.reshape(1, -1),
      Wp0, bp0.reshape(1, -1), Wp1, bp1.reshape(1, -1))


def kernel(node_ids, edge_index, batch, embed, W1, b1, g1, be1, W2, b2,
           g_bn, b_bn, Wp0, bp0, Wp1, bp1):
    nid_p = jnp.zeros((NP,), jnp.int32).at[:N].set(node_ids.astype(jnp.int32))
    src = edge_index[0].astype(jnp.int32)
    dst = edge_index[1].astype(jnp.int32)
    pad_iota = jnp.arange(EP, dtype=jnp.int32)
    src2d = (jnp.where(pad_iota < E, 0, pad_iota % N)
             .at[:E].set(src).reshape(NW * KW, EC))
    dst2d = (jnp.where(pad_iota < E, 0, N + pad_iota % (NP - N))
             .at[:E].set(dst).reshape(NW * KW, EC))
    x = _gather_call(embed, nid_p)
    agg = _edge_call(x, src2d, dst2d)
    batch2d = batch.astype(jnp.int32).reshape(N, 1)
    return _mlp_call(x, agg, batch2d, W1, b1, g1, be1, W2, b2,
                     g_bn, b_bn, Wp0, bp0, Wp1, bp1)


# speedup vs baseline: 10.9622x; 1.0087x over previous
import functools

import jax
import jax.numpy as jnp
from jax import lax
from jax.experimental import pallas as pl
from jax.experimental.pallas import tpu as pltpu
from jax.experimental.pallas import tpu_sc as plsc

N = 10000
E = 320000
D_IN = 128
D_H = 256
D_OUT = 128
G = 128

NC = 2
NS = 16
NW = NC * NS

NP = 10240
ROWS_W = NP // NW
ROWS_T = NP // NS

EC = 128
KW = 80
EP = NW * KW * EC


def _sc_mesh():
    return plsc.VectorSubcoreMesh(core_axis_name="c", subcore_axis_name="s")


def _gather_body(embed_hbm, nid_hbm, x_hbm, idx_v, rows_v, sem):
    wid = lax.axis_index("s") * NC + lax.axis_index("c")
    base = wid * ROWS_W
    pltpu.sync_copy(nid_hbm.at[pl.ds(base, ROWS_W)], idx_v)
    descs = []
    for off, sz in ((0, 128), (128, 128), (256, 64)):
        descs.append(
            pltpu.async_copy(
                embed_hbm.at[idx_v.at[pl.ds(off, sz)]],
                rows_v.at[pl.ds(off, sz)],
                sem,
            )
        )
    for d in descs:
        d.wait()
    pltpu.sync_copy(rows_v, x_hbm.at[pl.ds(base, ROWS_W)])


def _gather_call(embed, nid_p):
    k = pl.kernel(
        _gather_body,
        out_type=jax.ShapeDtypeStruct((NP, D_IN), jnp.float32),
        mesh=_sc_mesh(),
        scratch_types=[
            pltpu.VMEM((ROWS_W,), jnp.int32),
            pltpu.VMEM((ROWS_W, D_IN), jnp.float32),
            pltpu.SemaphoreType.DMA,
        ],
    )
    return k(embed, nid_p)


KH = KW // 2


def _edge_body(x_hbm, src_hbm, dst_hbm, agg_hbm,
               src_v, dst_v, buf0, buf1, gsem, ssem, agg_sh):
    cid = lax.axis_index("c")
    sid = lax.axis_index("s")
    wid = sid * NC + cid

    zv = jnp.zeros((16,), jnp.float32)

    def zr(i, carry):
        buf0[i // 8, pl.ds((i % 8) * 16, 16)] = zv
        return carry

    lax.fori_loop(0, EC * 8, zr, 0)
    for c in range(ROWS_T // EC):
        pltpu.sync_copy(buf0, agg_sh.at[pl.ds(sid * ROWS_T + c * EC, EC)])
    plsc.subcore_barrier()

    def drain_scatter(buf):
        pltpu.make_async_copy(buf, agg_sh.at[dst_v.at[0]], ssem).wait()

    for half in range(2):
        pltpu.sync_copy(src_hbm.at[pl.ds(wid * KW + half * KH, KH)], src_v)
        pltpu.sync_copy(dst_hbm.at[pl.ds(wid * KW + half * KH, KH)], dst_v)

        def body(i, carry):
            c0 = 2 * i

            @pl.when(i > 0)
            def _():
                drain_scatter(buf0)

            g0 = pltpu.async_copy(x_hbm.at[src_v.at[c0]], buf0, gsem)

            @pl.when(i > 0)
            def _():
                drain_scatter(buf1)

            g1 = pltpu.async_copy(x_hbm.at[src_v.at[c0 + 1]], buf1, gsem)
            g0.wait()
            pltpu.async_copy(buf0, agg_sh.at[dst_v.at[c0]], ssem, add=True)
            g1.wait()
            pltpu.async_copy(buf1, agg_sh.at[dst_v.at[c0 + 1]], ssem,
                             add=True)
            return carry

        lax.fori_loop(0, KH // 2, body, 0)
        drain_scatter(buf0)
        drain_scatter(buf1)
    plsc.subcore_barrier()
    pltpu.sync_copy(
        agg_sh.at[pl.ds(sid * ROWS_T, ROWS_T)],
        agg_hbm.at[cid, pl.ds(sid * ROWS_T, ROWS_T)],
    )


def _edge_call(x, src2d, dst2d):
    k = pl.kernel(
        _edge_body,
        out_type=jax.ShapeDtypeStruct((NC, NP, D_IN), jnp.float32),
        mesh=_sc_mesh(),
        scratch_types=[
            pltpu.VMEM((KH, EC), jnp.int32),
            pltpu.VMEM((KH, EC), jnp.int32),
            pltpu.VMEM((EC, D_IN), jnp.float32),
            pltpu.VMEM((EC, D_IN), jnp.float32),
            pltpu.SemaphoreType.DMA,
            pltpu.SemaphoreType.DMA,
            pltpu.VMEM_SHARED((NP, D_IN), jnp.float32),
        ],
    )
    return k(x, src2d, dst2d)


def _bn_cols(h, g, b):
    mu = jnp.mean(h, axis=0, keepdims=True)
    var = jnp.mean((h - mu) * (h - mu), axis=0, keepdims=True)
    return (h - mu) * lax.rsqrt(var + 1e-5) * g + b


def _dot_f32(a, b):
    return jnp.dot(a.astype(jnp.bfloat16), b.astype(jnp.bfloat16),
                   preferred_element_type=jnp.float32)


def _mlp_body(x_ref, agg_ref, batch_ref,
              w1_ref, b1_ref, g1_ref, be1_ref,
              w2_ref, b2_ref, gbn_ref, bbn_ref,
              wp0_ref, bp0_ref, wp1_ref, bp1_ref,
              out_ref):
    x = x_ref[0:N, :]
    h = x + agg_ref[0, 0:N, :] + agg_ref[1, 0:N, :]
    h1 = _dot_f32(h, w1_ref[...]) + b1_ref[...]
    h1 = _bn_cols(h1, g1_ref[...], be1_ref[...])
    h1 = jnp.maximum(h1, 0.0)
    h2 = _dot_f32(h1, w2_ref[...]) + b2_ref[...]
    h2 = _bn_cols(h2, gbn_ref[...], bbn_ref[...])
    h2 = jnp.maximum(h2, 0.0)
    score = (_dot_f32(x, wp0_ref[...]) + bp0_ref[...]
             + _dot_f32(h2, wp1_ref[...]) + bp1_ref[...])
    onehot = (batch_ref[...] ==
              lax.broadcasted_iota(jnp.int32, (N, G), 1)).astype(jnp.float32)
    sums = lax.dot_general(onehot.astype(jnp.bfloat16),
                           score.astype(jnp.bfloat16),
                           dimension_numbers=(((0,), (0,)), ((), ())),
                           preferred_element_type=jnp.float32)
    counts = jnp.sum(onehot, axis=0)
    out_ref[...] = sums / jnp.maximum(counts, 1.0)[:, None]


def _mlp_call(x, agg, batch2d, W1, b1, g1, be1, W2, b2, g_bn, b_bn,
              Wp0, bp0, Wp1, bp1):
    return pl.pallas_call(
        _mlp_body,
        out_shape=jax.ShapeDtypeStruct((G, D_OUT), jnp.float32),
    )(x, agg, batch2d,
      W1, b1.reshape(1, -1), g1.reshape(1, -1), be1.reshape(1, -1),
      W2, b2.reshape(1, -1), g_bn.reshape(1, -1), b_bn.reshape(1, -1),
      Wp0, bp0.reshape(1, -1), Wp1, bp1.reshape(1, -1))


def kernel(node_ids, edge_index, batch, embed, W1, b1, g1, be1, W2, b2,
           g_bn, b_bn, Wp0, bp0, Wp1, bp1):
    nid_p = (jnp.arange(NP, dtype=jnp.int32)
             .at[:N].set(node_ids.astype(jnp.int32)))
    src = edge_index[0].astype(jnp.int32)
    dst = edge_index[1].astype(jnp.int32)
    pad_iota = jnp.arange(EP, dtype=jnp.int32)
    src2d = (jnp.where(pad_iota < E, 0, pad_iota % N)
             .at[:E].set(src).reshape(NW * KW, EC))
    dst2d = (jnp.where(pad_iota < E, 0, N + pad_iota % (NP - N))
             .at[:E].set(dst).reshape(NW * KW, EC))
    x = _gather_call(embed, nid_p)
    agg = _edge_call(x, src2d, dst2d)
    batch2d = batch.astype(jnp.int32).reshape(N, 1)
    return _mlp_call(x, agg, batch2d, W1, b1, g1, be1, W2, b2,
                     g_bn, b_bn, Wp0, bp0, Wp1, bp1)


# speedup vs baseline: 11.9899x; 1.0937x over previous
import functools

import jax
import jax.numpy as jnp
from jax import lax
from jax.experimental import pallas as pl
from jax.experimental.pallas import tpu as pltpu
from jax.experimental.pallas import tpu_sc as plsc

N = 10000
E = 320000
D_IN = 128
D_H = 256
D_OUT = 128
G = 128

NC = 2
NS = 16
NW = NC * NS

NP = 10240
ROWS_W = NP // NW
ROWS_T = NP // NS

EC = 128
KW = 80
EP = NW * KW * EC


def _sc_mesh():
    return plsc.VectorSubcoreMesh(core_axis_name="c", subcore_axis_name="s")


def _gather_body(embed_hbm, nid_hbm, x_hbm, idx_v, rows_v, gsem, wsem):
    wid = lax.axis_index("s") * NC + lax.axis_index("c")
    base = jnp.minimum(wid * ROWS_W, N - ROWS_W)
    pltpu.sync_copy(nid_hbm.at[pl.ds(base, ROWS_W)], idx_v)
    chunks = ((0, 128), (128, 128), (256, 64))
    descs = [
        pltpu.async_copy(
            embed_hbm.at[idx_v.at[pl.ds(off, sz)]],
            rows_v.at[pl.ds(off, sz)],
            gsem,
        )
        for off, sz in chunks
    ]
    writes = []
    for (off, sz), d in zip(chunks, descs):
        d.wait()
        writes.append(
            pltpu.async_copy(rows_v.at[pl.ds(off, sz)],
                             x_hbm.at[pl.ds(base + off, sz)], wsem)
        )
    for w in writes:
        w.wait()


def _gather_call(embed, nid):
    k = pl.kernel(
        _gather_body,
        out_type=jax.ShapeDtypeStruct((N, D_IN), jnp.float32),
        mesh=_sc_mesh(),
        scratch_types=[
            pltpu.VMEM((ROWS_W,), jnp.int32),
            pltpu.VMEM((ROWS_W, D_IN), jnp.float32),
            pltpu.SemaphoreType.DMA,
            pltpu.SemaphoreType.DMA,
        ],
    )
    return k(embed, nid)


KH = KW // 2


def _edge_body(x_hbm, src_hbm, dst_hbm, agg_hbm,
               src_v, dst_v, buf0, buf1, gsem, ssem, agg_sh):
    cid = lax.axis_index("c")
    sid = lax.axis_index("s")
    wid = sid * NC + cid

    zv = jnp.zeros((16,), jnp.float32)

    def zr(i, carry):
        buf0[i // 8, pl.ds((i % 8) * 16, 16)] = zv
        return carry

    lax.fori_loop(0, EC * 8, zr, 0)
    for c in range(ROWS_T // EC):
        pltpu.sync_copy(buf0, agg_sh.at[pl.ds(sid * ROWS_T + c * EC, EC)])
    plsc.subcore_barrier()

    def drain_scatter(buf):
        pltpu.make_async_copy(buf, agg_sh.at[dst_v.at[0]], ssem).wait()

    for half in range(2):
        pltpu.sync_copy(src_hbm.at[pl.ds(wid * KW + half * KH, KH)], src_v)
        pltpu.sync_copy(dst_hbm.at[pl.ds(wid * KW + half * KH, KH)], dst_v)

        def body(i, carry):
            c0 = 2 * i

            @pl.when(i > 0)
            def _():
                drain_scatter(buf0)

            g0 = pltpu.async_copy(x_hbm.at[src_v.at[c0]], buf0, gsem)

            @pl.when(i > 0)
            def _():
                drain_scatter(buf1)

            g1 = pltpu.async_copy(x_hbm.at[src_v.at[c0 + 1]], buf1, gsem)
            g0.wait()
            pltpu.async_copy(buf0, agg_sh.at[dst_v.at[c0]], ssem, add=True)
            g1.wait()
            pltpu.async_copy(buf1, agg_sh.at[dst_v.at[c0 + 1]], ssem,
                             add=True)
            return carry

        lax.fori_loop(0, KH // 2, body, 0)
        drain_scatter(buf0)
        drain_scatter(buf1)
    plsc.subcore_barrier()
    pltpu.sync_copy(
        agg_sh.at[pl.ds(sid * ROWS_T, ROWS_T)],
        agg_hbm.at[cid, pl.ds(sid * ROWS_T, ROWS_T)],
    )


def _edge_call(x, src2d, dst2d):
    k = pl.kernel(
        _edge_body,
        out_type=jax.ShapeDtypeStruct((NC, NP, D_IN), jnp.float32),
        mesh=_sc_mesh(),
        scratch_types=[
            pltpu.VMEM((KH, EC), jnp.int32),
            pltpu.VMEM((KH, EC), jnp.int32),
            pltpu.VMEM((EC, D_IN), jnp.float32),
            pltpu.VMEM((EC, D_IN), jnp.float32),
            pltpu.SemaphoreType.DMA,
            pltpu.SemaphoreType.DMA,
            pltpu.VMEM_SHARED((NP, D_IN), jnp.float32),
        ],
    )
    return k(x, src2d, dst2d)


def _bn_cols(h, g, b):
    mu = jnp.mean(h, axis=0, keepdims=True)
    var = jnp.mean((h - mu) * (h - mu), axis=0, keepdims=True)
    return (h - mu) * lax.rsqrt(var + 1e-5) * g + b


def _dot_f32(a, b):
    return jnp.dot(a.astype(jnp.bfloat16), b.astype(jnp.bfloat16),
                   preferred_element_type=jnp.float32)


def _mlp_body(x_ref, agg_ref, batch_ref,
              w1_ref, b1_ref, g1_ref, be1_ref,
              w2_ref, b2_ref, gbn_ref, bbn_ref,
              wp0_ref, bp0_ref, wp1_ref, bp1_ref,
              out_ref):
    x = x_ref[0:N, :]
    h = x + agg_ref[0, 0:N, :] + agg_ref[1, 0:N, :]
    h1 = _dot_f32(h, w1_ref[...]) + b1_ref[...]
    h1 = _bn_cols(h1, g1_ref[...], be1_ref[...])
    h1 = jnp.maximum(h1, 0.0)
    h2 = _dot_f32(h1, w2_ref[...]) + b2_ref[...]
    h2 = _bn_cols(h2, gbn_ref[...], bbn_ref[...])
    h2 = jnp.maximum(h2, 0.0)
    score = (_dot_f32(x, wp0_ref[...]) + bp0_ref[...]
             + _dot_f32(h2, wp1_ref[...]) + bp1_ref[...])
    onehot = (batch_ref[...] ==
              lax.broadcasted_iota(jnp.int32, (N, G), 1)).astype(jnp.float32)
    sums = lax.dot_general(onehot.astype(jnp.bfloat16),
                           score.astype(jnp.bfloat16),
                           dimension_numbers=(((0,), (0,)), ((), ())),
                           preferred_element_type=jnp.float32)
    counts = jnp.sum(onehot, axis=0)
    out_ref[...] = sums / jnp.maximum(counts, 1.0)[:, None]


def _mlp_call(x, agg, batch2d, W1, b1, g1, be1, W2, b2, g_bn, b_bn,
              Wp0, bp0, Wp1, bp1):
    return pl.pallas_call(
        _mlp_body,
        out_shape=jax.ShapeDtypeStruct((G, D_OUT), jnp.float32),
    )(x, agg, batch2d,
      W1, b1.reshape(1, -1), g1.reshape(1, -1), be1.reshape(1, -1),
      W2, b2.reshape(1, -1), g_bn.reshape(1, -1), b_bn.reshape(1, -1),
      Wp0, bp0.reshape(1, -1), Wp1, bp1.reshape(1, -1))


def kernel(node_ids, edge_index, batch, embed, W1, b1, g1, be1, W2, b2,
           g_bn, b_bn, Wp0, bp0, Wp1, bp1):
    src = edge_index[0].astype(jnp.int32)
    dst = edge_index[1].astype(jnp.int32)
    pad_iota = jnp.arange(EP - E, dtype=jnp.int32)
    src2d = jnp.concatenate([src, pad_iota % N]).reshape(NW * KW, EC)
    dst2d = jnp.concatenate([dst, N + pad_iota % (NP - N)]).reshape(
        NW * KW, EC)
    x = _gather_call(embed, node_ids.astype(jnp.int32))
    agg = _edge_call(x, src2d, dst2d)
    batch2d = batch.astype(jnp.int32).reshape(N, 1)
    return _mlp_call(x, agg, batch2d, W1, b1, g1, be1, W2, b2,
                     g_bn, b_bn, Wp0, bp0, Wp1, bp1)


# speedup vs baseline: 12.0006x; 1.0009x over previous
import jax
import jax.numpy as jnp
from jax import lax
from jax.experimental import pallas as pl
from jax.experimental.pallas import tpu as pltpu
from jax.experimental.pallas import tpu_sc as plsc

N = 10000
E = 320000
D_IN = 128
D_H = 256
D_OUT = 128
G = 128

NC = 2
NS = 16
NW = NC * NS

NP = 10240
ROWS_W = NP // NW
ROWS_T = NP // NS

EC = 128
KW = 80
EP = NW * KW * EC


def _sc_mesh():
    return plsc.VectorSubcoreMesh(core_axis_name="c", subcore_axis_name="s")


def _gather_body(embed_hbm, nid_hbm, x_hbm, idx_v, rows_v, gsem, wsem):
    wid = lax.axis_index("s") * NC + lax.axis_index("c")
    base = jnp.minimum(wid * ROWS_W, N - ROWS_W)
    pltpu.sync_copy(nid_hbm.at[pl.ds(base, ROWS_W)], idx_v)
    chunks = ((0, 128), (128, 128), (256, 64))
    descs = [
        pltpu.async_copy(
            embed_hbm.at[idx_v.at[pl.ds(off, sz)]],
            rows_v.at[pl.ds(off, sz)],
            gsem,
        )
        for off, sz in chunks
    ]
    writes = []
    for (off, sz), d in zip(chunks, descs):
        d.wait()
        writes.append(
            pltpu.async_copy(rows_v.at[pl.ds(off, sz)],
                             x_hbm.at[pl.ds(base + off, sz)], wsem)
        )
    for w in writes:
        w.wait()


def _gather_call(embed, nid):
    k = pl.kernel(
        _gather_body,
        out_type=jax.ShapeDtypeStruct((N, D_IN), jnp.float32),
        mesh=_sc_mesh(),
        scratch_types=[
            pltpu.VMEM((ROWS_W,), jnp.int32),
            pltpu.VMEM((ROWS_W, D_IN), jnp.float32),
            pltpu.SemaphoreType.DMA,
            pltpu.SemaphoreType.DMA,
        ],
    )
    return k(embed, nid)


KH = KW // 2


def _edge_body(x_hbm, src_hbm, dst_hbm, agg_hbm,
               src_v, dst_v, buf0, buf1, gsem, ssem, agg_sh):
    cid = lax.axis_index("c")
    sid = lax.axis_index("s")
    wid = sid * NC + cid

    zv = jnp.zeros((16,), jnp.float32)

    def zr(i, carry):
        buf0[i // 8, pl.ds((i % 8) * 16, 16)] = zv
        return carry

    lax.fori_loop(0, EC * 8, zr, 0)
    for c in range(ROWS_T // EC):
        pltpu.sync_copy(buf0, agg_sh.at[pl.ds(sid * ROWS_T + c * EC, EC)])
    plsc.subcore_barrier()

    def drain_scatter(buf):
        pltpu.make_async_copy(buf, agg_sh.at[dst_v.at[0]], ssem).wait()

    for half in range(2):
        pltpu.sync_copy(src_hbm.at[pl.ds(wid * KW + half * KH, KH)], src_v)
        pltpu.sync_copy(dst_hbm.at[pl.ds(wid * KW + half * KH, KH)], dst_v)

        def body(i, carry):
            c0 = 2 * i

            @pl.when(i > 0)
            def _():
                drain_scatter(buf0)

            g0 = pltpu.async_copy(x_hbm.at[src_v.at[c0]], buf0, gsem)

            @pl.when(i > 0)
            def _():
                drain_scatter(buf1)

            g1 = pltpu.async_copy(x_hbm.at[src_v.at[c0 + 1]], buf1, gsem)
            g0.wait()
            pltpu.async_copy(buf0, agg_sh.at[dst_v.at[c0]], ssem, add=True)
            g1.wait()
            pltpu.async_copy(buf1, agg_sh.at[dst_v.at[c0 + 1]], ssem,
                             add=True)
            return carry

        lax.fori_loop(0, KH // 2, body, 0)
        drain_scatter(buf0)
        drain_scatter(buf1)
    plsc.subcore_barrier()
    pltpu.sync_copy(
        agg_sh.at[pl.ds(sid * ROWS_T, ROWS_T)],
        agg_hbm.at[cid, pl.ds(sid * ROWS_T, ROWS_T)],
    )


def _edge_call(x, src2d, dst2d):
    k = pl.kernel(
        _edge_body,
        out_type=jax.ShapeDtypeStruct((NC, NP, D_IN), jnp.float32),
        mesh=_sc_mesh(),
        scratch_types=[
            pltpu.VMEM((KH, EC), jnp.int32),
            pltpu.VMEM((KH, EC), jnp.int32),
            pltpu.VMEM((EC, D_IN), jnp.float32),
            pltpu.VMEM((EC, D_IN), jnp.float32),
            pltpu.SemaphoreType.DMA,
            pltpu.SemaphoreType.DMA,
            pltpu.VMEM_SHARED((NP, D_IN), jnp.float32),
        ],
    )
    return k(x, src2d, dst2d)


def _bn_cols(h, g, b):
    mu = jnp.mean(h, axis=0, keepdims=True)
    var = jnp.mean((h - mu) * (h - mu), axis=0, keepdims=True)
    return (h - mu) * lax.rsqrt(var + 1e-5) * g + b


def _dot_f32(a, b):
    return jnp.dot(a.astype(jnp.bfloat16), b.astype(jnp.bfloat16),
                   preferred_element_type=jnp.float32)


def _mlp_body(x_ref, agg_ref, batch_ref,
              w1_ref, b1_ref, g1_ref, be1_ref,
              w2_ref, b2_ref, gbn_ref, bbn_ref,
              wp0_ref, bp0_ref, wp1_ref, bp1_ref,
              out_ref):
    x = x_ref[0:N, :]
    h = x + agg_ref[0, 0:N, :] + agg_ref[1, 0:N, :]
    h1 = _dot_f32(h, w1_ref[...]) + b1_ref[...]
    h1 = _bn_cols(h1, g1_ref[...], be1_ref[...])
    h1 = jnp.maximum(h1, 0.0)
    h2 = _dot_f32(h1, w2_ref[...]) + b2_ref[...]
    h2 = _bn_cols(h2, gbn_ref[...], bbn_ref[...])
    h2 = jnp.maximum(h2, 0.0)
    score = (_dot_f32(x, wp0_ref[...]) + bp0_ref[...]
             + _dot_f32(h2, wp1_ref[...]) + bp1_ref[...])
    onehot = (batch_ref[...] ==
              lax.broadcasted_iota(jnp.int32, (N, G), 1)).astype(jnp.float32)
    sums = lax.dot_general(onehot.astype(jnp.bfloat16),
                           score.astype(jnp.bfloat16),
                           dimension_numbers=(((0,), (0,)), ((), ())),
                           preferred_element_type=jnp.float32)
    counts = jnp.sum(onehot, axis=0)
    out_ref[...] = sums / jnp.maximum(counts, 1.0)[:, None]


def _mlp_call(x, agg, batch2d, W1, b1, g1, be1, W2, b2, g_bn, b_bn,
              Wp0, bp0, Wp1, bp1):
    return pl.pallas_call(
        _mlp_body,
        out_shape=jax.ShapeDtypeStruct((G, D_OUT), jnp.float32),
    )(x, agg, batch2d,
      W1, b1.reshape(1, -1), g1.reshape(1, -1), be1.reshape(1, -1),
      W2, b2.reshape(1, -1), g_bn.reshape(1, -1), b_bn.reshape(1, -1),
      Wp0, bp0.reshape(1, -1), Wp1, bp1.reshape(1, -1))


def kernel(node_ids, edge_index, batch, embed, W1, b1, g1, be1, W2, b2,
           g_bn, b_bn, Wp0, bp0, Wp1, bp1):
    src = edge_index[0].astype(jnp.int32)
    dst = edge_index[1].astype(jnp.int32)
    pad_iota = jnp.arange(EP - E, dtype=jnp.int32)
    src2d = jnp.concatenate([src, pad_iota % N]).reshape(NW * KW, EC)
    dst2d = jnp.concatenate([dst, N + pad_iota % (NP - N)]).reshape(
        NW * KW, EC)
    x = _gather_call(embed, node_ids.astype(jnp.int32))
    agg = _edge_call(x, src2d, dst2d)
    batch2d = batch.astype(jnp.int32).reshape(N, 1)
    return _mlp_call(x, agg, batch2d, W1, b1, g1, be1, W2, b2,
                     g_bn, b_bn, Wp0, bp0, Wp1, bp1)
